# baseline (speedup 1.0000x reference)
import functools

import jax
import jax.numpy as jnp
from jax import lax
from jax.experimental import pallas as pl
from jax.experimental.pallas import tpu as pltpu
from jax.experimental.pallas import tpu_sc as plsc

B = 16384
D = 64
L = 16
NC = 2
NS = 16
NW = NC * NS

NB_USERS = 1000000
NB_ITEMS = 100000
UQ = (NB_USERS + 127) // 128
IQ = (NB_ITEMS + 127) // 128
UPT = (UQ + NW - 1) // NW
IPT = (IQ + NW - 1) // NW
WC = 3
UWIN = (UPT + WC - 1) // WC
IWIN = (IPT + WC - 1) // WC
NSCAN = B // L
LCAP = B + 2 * L
PREF = 128
BDUMP = B

_DNUMS = lax.GatherDimensionNumbers(
    offset_dims=(), collapsed_slice_dims=(0,), start_index_map=(0,))


def _shuffle(p, perm):
    return lax.gather(p, perm, _DNUMS, (1,),
                      mode=lax.GatherScatterMode.PROMISE_IN_BOUNDS)


def _lane():
    return lax.broadcasted_iota(jnp.int32, (L,), 0)


def _lanesum(p):
    lane = _lane()
    for s in range(4):
        p = p + _shuffle(p, (lane ^ (8 >> s))[:, None])
    return p


def _splat_at(ref, e):
    return plsc.load_gather(ref, [jnp.zeros((L,), jnp.int32) + e])


def _scan_matches(idx_v, vallist, blist, lo, hi):
    lane = _lane()

    def step(i, cnt):
        v = idx_v[pl.ds(i * L, L)]
        q = v >> 7
        m = (q >= lo) & (q < hi)
        plsc.store_compressed(vallist.at[pl.ds(cnt, L)], v, mask=m)
        plsc.store_compressed(blist.at[pl.ds(cnt, L)], (i * L) + lane, mask=m)
        return cnt + jnp.max(plsc.all_reduce_population_count(m))

    return lax.fori_loop(0, NSCAN, step, jnp.int32(0))


def _window_scan(vallist, blist, winlist, cnt, wlo):
    lane = _lane()
    nv = (cnt + L - 1) // L

    def step(g, wcnt):
        v = vallist[pl.ds(g * L, L)]
        b = blist[pl.ds(g * L, L)]
        q = v >> 7
        m = (q >= wlo) & (q < wlo + WC) & ((g * L + lane) < cnt)
        packed = ((q - wlo) << 22) | ((v & 127) << 15) | b
        plsc.store_compressed(winlist.at[pl.ds(wcnt, L)], packed, mask=m)
        return wcnt + jnp.max(plsc.all_reduce_population_count(m))

    wcnt = lax.fori_loop(0, nv, step, jnp.int32(0))
    winlist[pl.ds(wcnt, L)] = jnp.full((L,), BDUMP, jnp.int32)
    return wcnt


def _minor_of(ps):
    return ((ps >> 22) << 7) | ((ps >> 15) & 127)


def _item_body(item_hbm, iembT_hbm, w_hbm, itemw_hbm,
               iidx_v, slist, blist, winlist, winbuf0, winbuf1, stag, w_v,
               sw0, sw1, sem_s):
    wid = lax.axis_index("s") * NC + lax.axis_index("c")
    lo = wid * IPT
    hi = jnp.minimum(lo + IPT, IQ)

    pltpu.sync_copy(item_hbm, iidx_v)
    pltpu.sync_copy(w_hbm, w_v)
    wv = tuple(w_v[pl.ds(j * L, L)] for j in range(4))
    lane = _lane()
    cvec = tuple(lane + j * L for j in range(4))

    cnt = _scan_matches(iidx_v, slist, blist, lo, hi)

    def fire_win(w, buf, sem):
        for ci in range(WC):
            q = jnp.minimum(lo + w * WC + ci, IQ - 1)
            pltpu.async_copy(
                iembT_hbm.at[:, pl.ds(pl.multiple_of(q * 128, 128), 128)],
                buf.at[:, pl.ds(ci * 128, 128)], sem)

    def one(e, buf):
        ps = _splat_at(winlist, e)
        minor = _minor_of(ps)
        for j in range(4):
            g = plsc.load_gather(buf, [cvec[j], minor]) * wv[j]
            stag[pl.ds(e * D + j * L, L)] = g
        b = jnp.max(ps) & 32767
        pltpu.async_copy(stag.at[pl.ds(e * D, D)],
                         itemw_hbm.at[pl.ds(b * D, D)], sem_s)

    def process(w, buf, sem):
        wlo = lo + w * WC
        wcnt = _window_scan(slist, blist, winlist, cnt, wlo)
        npf = jnp.minimum(wcnt, PREF)
        pltpu.make_async_copy(iembT_hbm.at[:, pl.ds(0, WC * 128)], buf,
                              sem).wait()
        nhalf = (npf + 1) >> 1

        def fast(e2, _):
            one(2 * e2, buf)
            one(2 * e2 + 1, buf)
            return _

        lax.fori_loop(0, nhalf, fast, 0)

        def slow(e, _):
            ps = _splat_at(winlist, e)
            minor = _minor_of(ps)
            for j in range(4):
                g = plsc.load_gather(buf, [cvec[j], minor]) * wv[j]
                stag[pl.ds(j * L, L)] = g
            b = jnp.max(ps) & 32767
            pltpu.async_copy(stag.at[pl.ds(0, D)],
                             itemw_hbm.at[pl.ds(b * D, D)], sem_s).wait()
            return _

        lax.fori_loop(npf, wcnt, slow, 0)

        def drain(e, _):
            pltpu.make_async_copy(itemw_hbm.at[pl.ds(0, D)],
                                  stag.at[pl.ds(0, D)], sem_s).wait()
            return _

        lax.fori_loop(0, 2 * nhalf, drain, 0)

    fire_win(0, winbuf0, sw0)
    fire_win(1, winbuf1, sw1)

    def pair(i, _):
        process(2 * i, winbuf0, sw0)

        @pl.when(2 * i + 2 < IWIN)
        def _f0():
            fire_win(2 * i + 2, winbuf0, sw0)

        process(2 * i + 1, winbuf1, sw1)

        @pl.when(2 * i + 3 < IWIN)
        def _f1():
            fire_win(2 * i + 3, winbuf1, sw1)

        return _

    lax.fori_loop(0, IWIN // 2, pair, 0)
    if IWIN % 2:
        process(IWIN - 1, winbuf0, sw0)


def _user_body(user_hbm, uembT_hbm, itemw_hbm, bias_hbm, outw_hbm,
               uidx_v, rlist, blist, winlist, winbuf0, winbuf1, irows, vals,
               bias_v, sw0, sw1, sem_r, sem_o):
    wid = lax.axis_index("s") * NC + lax.axis_index("c")
    lo = wid * UPT
    hi = jnp.minimum(lo + UPT, UQ)

    pltpu.sync_copy(user_hbm, uidx_v)
    pltpu.sync_copy(bias_hbm, bias_v)
    bias = bias_v[...]
    lane = _lane()
    cvec = tuple(lane + j * L for j in range(4))

    cnt = _scan_matches(uidx_v, rlist, blist, lo, hi)

    def fire_win(w, buf, sem):
        for ci in range(WC):
            q = jnp.minimum(lo + w * WC + ci, UQ - 1)
            pltpu.async_copy(
                uembT_hbm.at[:, pl.ds(pl.multiple_of(q * 128, 128), 128)],
                buf.at[:, pl.ds(ci * 128, 128)], sem)

    def pref_one(e):
        ps = _splat_at(winlist, e)
        b = jnp.max(ps) & 32767
        pltpu.async_copy(itemw_hbm.at[pl.ds(b * D, D)],
                         irows.at[pl.ds(e * D, D)], sem_r)

    def one(e, buf):
        ps = _splat_at(winlist, e)
        minor = _minor_of(ps)
        acc = None
        for j in range(4):
            t = (plsc.load_gather(buf, [cvec[j], minor])
                 * irows[pl.ds(e * D + j * L, L)])
            acc = t if acc is None else acc + t
        val = _lanesum(acc) + bias
        vals[pl.ds(e * L, L)] = val
        b = jnp.max(ps) & 32767
        pltpu.async_copy(vals.at[pl.ds(e * L, L)],
                         outw_hbm.at[pl.ds(b * L, L)], sem_o)

    def process(w, buf, sem):
        wlo = lo + w * WC
        wcnt = _window_scan(rlist, blist, winlist, cnt, wlo)
        npf = jnp.minimum(wcnt, PREF)
        nhalf = (npf + 1) >> 1

        def pref(e2, _):
            pref_one(2 * e2)
            pref_one(2 * e2 + 1)
            return _

        lax.fori_loop(0, nhalf, pref, 0)
        pltpu.make_async_copy(uembT_hbm.at[:, pl.ds(0, WC * 128)], buf,
                              sem).wait()

        def drain_r(e, _):
            pltpu.make_async_copy(itemw_hbm.at[pl.ds(0, D)],
                                  irows.at[pl.ds(0, D)], sem_r).wait()
            return _

        lax.fori_loop(0, 2 * nhalf, drain_r, 0)

        def fast(e2, _):
            one(2 * e2, buf)
            one(2 * e2 + 1, buf)
            return _

        lax.fori_loop(0, nhalf, fast, 0)

        def slow(e, _):
            ps = _splat_at(winlist, e)
            minor = _minor_of(ps)
            b = jnp.max(ps) & 32767
            pltpu.async_copy(itemw_hbm.at[pl.ds(b * D, D)],
                             irows.at[pl.ds(0, D)], sem_r).wait()
            acc = None
            for j in range(4):
                t = (plsc.load_gather(buf, [cvec[j], minor])
                     * irows[pl.ds(j * L, L)])
                acc = t if acc is None else acc + t
            val = _lanesum(acc) + bias
            vals[pl.ds(0, L)] = val
            pltpu.async_copy(vals.at[pl.ds(0, L)],
                             outw_hbm.at[pl.ds(b * L, L)], sem_o).wait()
            return _

        lax.fori_loop(npf, wcnt, slow, 0)

        def drain_o(e, _):
            pltpu.make_async_copy(outw_hbm.at[pl.ds(0, L)],
                                  vals.at[pl.ds(0, L)], sem_o).wait()
            return _

        lax.fori_loop(0, 2 * nhalf, drain_o, 0)

    fire_win(0, winbuf0, sw0)
    fire_win(1, winbuf1, sw1)

    def pair(i, _):
        process(2 * i, winbuf0, sw0)

        @pl.when(2 * i + 2 < UWIN)
        def _f0():
            fire_win(2 * i + 2, winbuf0, sw0)

        process(2 * i + 1, winbuf1, sw1)

        @pl.when(2 * i + 3 < UWIN)
        def _f1():
            fire_win(2 * i + 3, winbuf1, sw1)

        return _

    lax.fori_loop(0, UWIN // 2, pair, 0)
    if UWIN % 2:
        process(UWIN - 1, winbuf0, sw0)


@jax.jit
def kernel(user, item, mf_user_embed, mf_item_embed, final_w, final_b):
    uembT = mf_user_embed.T
    iembT = mf_item_embed.T
    w_flat = final_w.reshape(D)
    bias16 = jnp.tile(final_b.reshape(1), L)
    mesh = plsc.VectorSubcoreMesh(core_axis_name="c", subcore_axis_name="s")
    cp = pltpu.CompilerParams(use_tc_tiling_on_sc=True,
                              needs_layout_passes=False)

    item_run = functools.partial(
        pl.kernel, mesh=mesh, compiler_params=cp,
        out_type=jax.ShapeDtypeStruct(((B + L) * D,), jnp.float32),
        scratch_types=[
            pltpu.VMEM((B,), jnp.int32),
            pltpu.VMEM((LCAP,), jnp.int32),
            pltpu.VMEM((LCAP,), jnp.int32),
            pltpu.VMEM((LCAP,), jnp.int32),
            pltpu.VMEM((D, WC * 128), jnp.float32),
            pltpu.VMEM((D, WC * 128), jnp.float32),
            pltpu.VMEM(((PREF + 2) * D,), jnp.float32),
            pltpu.VMEM((D,), jnp.float32),
            pltpu.SemaphoreType.DMA,
            pltpu.SemaphoreType.DMA,
            pltpu.SemaphoreType.DMA,
        ],
    )(_item_body)
    itemw = item_run(item.astype(jnp.int32), iembT, w_flat)

    user_run = functools.partial(
        pl.kernel, mesh=mesh, compiler_params=cp,
        out_type=jax.ShapeDtypeStruct(((B + L) * L,), jnp.float32),
        scratch_types=[
            pltpu.VMEM((B,), jnp.int32),
            pltpu.VMEM((LCAP,), jnp.int32),
            pltpu.VMEM((LCAP,), jnp.int32),
            pltpu.VMEM((LCAP,), jnp.int32),
            pltpu.VMEM((D, WC * 128), jnp.float32),
            pltpu.VMEM((D, WC * 128), jnp.float32),
            pltpu.VMEM(((PREF + 2) * D,), jnp.float32),
            pltpu.VMEM(((PREF + 2) * L,), jnp.float32),
            pltpu.VMEM((L,), jnp.float32),
            pltpu.SemaphoreType.DMA,
            pltpu.SemaphoreType.DMA,
            pltpu.SemaphoreType.DMA,
            pltpu.SemaphoreType.DMA,
        ],
    )(_user_body)
    outw = user_run(user.astype(jnp.int32), uembT, itemw, bias16)

    return outw.reshape(B + L, L)[:B, 0:1]


# speedup vs baseline: 1.0854x; 1.0854x over previous
import functools

import jax
import jax.numpy as jnp
from jax import lax
from jax.experimental import pallas as pl
from jax.experimental.pallas import tpu as pltpu
from jax.experimental.pallas import tpu_sc as plsc

B = 16384
D = 64
L = 16
NC = 2
NS = 16
NW = NC * NS

NB_USERS = 1000000
NB_ITEMS = 100000
UQ = (NB_USERS + 127) // 128
IQ = (NB_ITEMS + 127) // 128
UPT = (UQ + NW - 1) // NW
IPT = (IQ + NW - 1) // NW
WC = 3
UWIN = (UPT + WC - 1) // WC
IWIN = (IPT + WC - 1) // WC
NSCAN = B // L
NSUB = 8
WPS = 11
SUBW = WPS * WC
LCAP = B + L
PREF = 128

_DNUMS = lax.GatherDimensionNumbers(
    offset_dims=(), collapsed_slice_dims=(0,), start_index_map=(0,))


def _shuffle(p, perm):
    return lax.gather(p, perm, _DNUMS, (1,),
                      mode=lax.GatherScatterMode.PROMISE_IN_BOUNDS)


def _lane():
    return lax.broadcasted_iota(jnp.int32, (L,), 0)


def _lanesum(p):
    lane = _lane()
    for s in range(4):
        p = p + _shuffle(p, (lane ^ (8 >> s))[:, None])
    return p


def _splat_at(ref, e):
    return plsc.load_gather(ref, [jnp.zeros((L,), jnp.int32) + e])


def _scan_matches(idx_v, vallist, blist, lo, hi):
    lane = _lane()

    def step(i, cnt):
        v = idx_v[pl.ds(i * L, L)]
        q = v >> 7
        m = (q >= lo) & (q < hi)
        plsc.store_compressed(vallist.at[pl.ds(cnt, L)], v, mask=m)
        plsc.store_compressed(blist.at[pl.ds(cnt, L)], (i * L) + lane, mask=m)
        return cnt + jnp.max(plsc.all_reduce_population_count(m))

    return lax.fori_loop(0, NSCAN, step, jnp.int32(0))


def _window_scan(vallist, blist, winlist, cnt, wlo):
    lane = _lane()
    nv = (cnt + L - 1) // L

    def step(g, wcnt):
        v = vallist[pl.ds(g * L, L)]
        b = blist[pl.ds(g * L, L)]
        q = v >> 7
        m = (q >= wlo) & (q < wlo + WC) & ((g * L + lane) < cnt)
        packed = ((q - wlo) << 21) | ((v & 127) << 14) | b
        plsc.store_compressed(winlist.at[pl.ds(wcnt, L)], packed, mask=m)
        return wcnt + jnp.max(plsc.all_reduce_population_count(m))

    return lax.fori_loop(0, nv, step, jnp.int32(0))


def _minor_of(ps):
    return ((ps >> 21) << 7) | ((ps >> 14) & 127)


_CVEC = None


def _item_body(item_hbm, iembT_hbm, w_hbm, itemw_hbm,
               iidx_v, slist, blist, winlist, winbuf, stag, w_v,
               sem_w, sem_s):
    wid = lax.axis_index("s") * NC + lax.axis_index("c")
    lo = wid * IPT
    hi = jnp.minimum(lo + IPT, IQ)

    pltpu.sync_copy(item_hbm, iidx_v)
    pltpu.sync_copy(w_hbm, w_v)
    wv = tuple(w_v[pl.ds(j * L, L)] for j in range(4))
    lane = _lane()
    cvec = tuple(lane + j * L for j in range(4))

    cnt = _scan_matches(iidx_v, slist, blist, lo, hi)

    def window(w, _):
        wlo = lo + w * WC
        for ci in range(WC):
            q = jnp.minimum(wlo + ci, IQ - 1)
            pltpu.async_copy(
                iembT_hbm.at[:, pl.ds(pl.multiple_of(q * 128, 128), 128)],
                winbuf.at[:, pl.ds(ci * 128, 128)], sem_w)
        wcnt = _window_scan(slist, blist, winlist, cnt, wlo)
        npf = jnp.minimum(wcnt, PREF)
        pltpu.make_async_copy(iembT_hbm.at[:, pl.ds(0, WC * 128)], winbuf,
                              sem_w).wait()

        def fast(e, _):
            ps = _splat_at(winlist, e)
            minor = _minor_of(ps)
            for j in range(4):
                g = plsc.load_gather(winbuf, [cvec[j], minor]) * wv[j]
                stag[pl.ds(e * D + j * L, L)] = g
            b = jnp.max(ps) & 16383
            pltpu.async_copy(stag.at[pl.ds(e * D, D)],
                             itemw_hbm.at[pl.ds(b * D, D)], sem_s)
            return _

        lax.fori_loop(0, npf, fast, 0)

        def slow(e, _):
            ps = _splat_at(winlist, e)
            minor = _minor_of(ps)
            for j in range(4):
                g = plsc.load_gather(winbuf, [cvec[j], minor]) * wv[j]
                stag[pl.ds(j * L, L)] = g
            b = jnp.max(ps) & 16383
            pltpu.async_copy(stag.at[pl.ds(0, D)],
                             itemw_hbm.at[pl.ds(b * D, D)], sem_s).wait()
            return _

        lax.fori_loop(npf, wcnt, slow, 0)

        def drain(e, _):
            pltpu.make_async_copy(itemw_hbm.at[pl.ds(0, D)],
                                  stag.at[pl.ds(0, D)], sem_s).wait()
            return _

        lax.fori_loop(0, npf, drain, 0)
        return _

    lax.fori_loop(0, IWIN, window, 0)


def _user_body(user_hbm, uembT_hbm, itemw_hbm, bias_hbm, outw_hbm,
               uidx_v, plist, plist2, winlist, winbuf0, winbuf1, irows, vals,
               bias_v, meta_v, sw0, sw1, sem_r, sem_o):
    wid = lax.axis_index("s") * NC + lax.axis_index("c")
    lo = wid * UPT
    hi = jnp.minimum(lo + UPT, UQ)

    pltpu.sync_copy(user_hbm, uidx_v)
    pltpu.sync_copy(bias_hbm, bias_v)
    bias = bias_v[...]
    lane = _lane()
    cvec = tuple(lane + j * L for j in range(4))

    def scan_step(i, cnt):
        v = uidx_v[pl.ds(i * L, L)]
        q = v >> 7
        m = (q >= lo) & (q < hi)
        packed = ((q - lo) << 21) | ((v & 127) << 14) | ((i * L) + lane)
        plsc.store_compressed(plist.at[pl.ds(cnt, L)], packed, mask=m)
        return cnt + jnp.max(plsc.all_reduce_population_count(m))

    cnt = lax.fori_loop(0, NSCAN, scan_step, jnp.int32(0))
    nv = (cnt + L - 1) // L

    def count_step(g, counts):
        p = plist[pl.ds(g * L, L)]
        sub = (p >> 21) // SUBW
        ok = (g * L + lane) < cnt
        return tuple(
            counts[s]
            + jnp.max(plsc.all_reduce_population_count((sub == s) & ok))
            for s in range(NSUB))

    counts = lax.fori_loop(0, nv, count_step,
                           tuple(jnp.int32(0) for _ in range(NSUB)))
    offs = []
    acc = jnp.int32(0)
    for s in range(NSUB):
        offs.append(acc)
        acc = acc + counts[s]
    meta = jnp.zeros((L,), jnp.int32)
    for s in range(NSUB):
        meta = jnp.where(lane == s, offs[s], meta)
        meta = jnp.where(lane == (s + NSUB), counts[s], meta)
    meta_v[...] = meta

    def place_step(g, curs):
        p = plist[pl.ds(g * L, L)]
        sub = (p >> 21) // SUBW
        ok = (g * L + lane) < cnt
        out = []
        for s in range(NSUB):
            m = (sub == s) & ok
            plsc.store_compressed(plist2.at[pl.ds(curs[s], L)], p, mask=m)
            out.append(curs[s]
                       + jnp.max(plsc.all_reduce_population_count(m)))
        return tuple(out)

    lax.fori_loop(0, nv, place_step, tuple(offs))

    def fire_win(w, buf, sem):
        for ci in range(WC):
            q = jnp.minimum(lo + w * WC + ci, UQ - 1)
            pltpu.async_copy(
                uembT_hbm.at[:, pl.ds(pl.multiple_of(q * 128, 128), 128)],
                buf.at[:, pl.ds(ci * 128, 128)], sem)

    def window_scan_sub(w):
        s = w // WPS
        off_s = jnp.max(_splat_at(meta_v, s))
        cnt_s = jnp.max(_splat_at(meta_v, s + NSUB))
        wrel = w * WC
        nvs = (cnt_s + L - 1) // L

        def step(g, wcnt):
            p = plist2[pl.ds(off_s + g * L, L)]
            qr = p >> 21
            m = ((qr >= wrel) & (qr < wrel + WC)
                 & ((g * L + lane) < cnt_s))
            packed = ((qr - wrel) << 21) | (p & 2097151)
            plsc.store_compressed(winlist.at[pl.ds(wcnt, L)], packed, mask=m)
            return wcnt + jnp.max(plsc.all_reduce_population_count(m))

        return lax.fori_loop(0, nvs, step, jnp.int32(0))

    def pref_one(e):
        ps = _splat_at(winlist, e)
        b = jnp.max(ps) & 16383
        pltpu.async_copy(itemw_hbm.at[pl.ds(b * D, D)],
                         irows.at[pl.ds(e * D, D)], sem_r)

    def one(e, buf):
        ps = _splat_at(winlist, e)
        minor = _minor_of(ps)
        acc = None
        for j in range(4):
            t = (plsc.load_gather(buf, [cvec[j], minor])
                 * irows[pl.ds(e * D + j * L, L)])
            acc = t if acc is None else acc + t
        val = _lanesum(acc) + bias
        vals[pl.ds(e * L, L)] = val
        b = jnp.max(ps) & 16383
        pltpu.async_copy(vals.at[pl.ds(e * L, L)],
                         outw_hbm.at[pl.ds(b * L, L)], sem_o)

    def process(w, buf, sem):
        wcnt = window_scan_sub(w)
        npf = jnp.minimum(wcnt, PREF)

        def pref(e, _):
            pref_one(e)
            return _

        lax.fori_loop(0, npf, pref, 0)
        pltpu.make_async_copy(uembT_hbm.at[:, pl.ds(0, WC * 128)], buf,
                              sem).wait()

        def drain_r(e, _):
            pltpu.make_async_copy(itemw_hbm.at[pl.ds(0, D)],
                                  irows.at[pl.ds(0, D)], sem_r).wait()
            return _

        lax.fori_loop(0, npf, drain_r, 0)

        def fast(e, _):
            one(e, buf)
            return _

        lax.fori_loop(0, npf, fast, 0)

        def slow(e, _):
            ps = _splat_at(winlist, e)
            minor = _minor_of(ps)
            b = jnp.max(ps) & 16383
            pltpu.async_copy(itemw_hbm.at[pl.ds(b * D, D)],
                             irows.at[pl.ds(0, D)], sem_r).wait()
            acc = None
            for j in range(4):
                t = (plsc.load_gather(buf, [cvec[j], minor])
                     * irows[pl.ds(j * L, L)])
                acc = t if acc is None else acc + t
            val = _lanesum(acc) + bias
            vals[pl.ds(0, L)] = val
            pltpu.async_copy(vals.at[pl.ds(0, L)],
                             outw_hbm.at[pl.ds(b * L, L)], sem_o).wait()
            return _

        lax.fori_loop(npf, wcnt, slow, 0)

        def drain_o(e, _):
            pltpu.make_async_copy(outw_hbm.at[pl.ds(0, L)],
                                  vals.at[pl.ds(0, L)], sem_o).wait()
            return _

        lax.fori_loop(0, npf, drain_o, 0)

    fire_win(0, winbuf0, sw0)
    fire_win(1, winbuf1, sw1)

    def pair(i, _):
        process(2 * i, winbuf0, sw0)

        @pl.when(2 * i + 2 < UWIN)
        def _f0():
            fire_win(2 * i + 2, winbuf0, sw0)

        process(2 * i + 1, winbuf1, sw1)

        @pl.when(2 * i + 3 < UWIN)
        def _f1():
            fire_win(2 * i + 3, winbuf1, sw1)

        return _

    lax.fori_loop(0, (UWIN + 1) // 2, pair, 0)


@jax.jit
def kernel(user, item, mf_user_embed, mf_item_embed, final_w, final_b):
    uembT = mf_user_embed.T
    iembT = mf_item_embed.T
    w_flat = final_w.reshape(D)
    bias16 = jnp.tile(final_b.reshape(1), L)
    mesh = plsc.VectorSubcoreMesh(core_axis_name="c", subcore_axis_name="s")
    cp = pltpu.CompilerParams(use_tc_tiling_on_sc=True,
                              needs_layout_passes=False)

    item_run = functools.partial(
        pl.kernel, mesh=mesh, compiler_params=cp,
        out_type=jax.ShapeDtypeStruct((B * D,), jnp.float32),
        scratch_types=[
            pltpu.VMEM((B,), jnp.int32),
            pltpu.VMEM((LCAP,), jnp.int32),
            pltpu.VMEM((LCAP,), jnp.int32),
            pltpu.VMEM((LCAP,), jnp.int32),
            pltpu.VMEM((D, WC * 128), jnp.float32),
            pltpu.VMEM((PREF * D,), jnp.float32),
            pltpu.VMEM((D,), jnp.float32),
            pltpu.SemaphoreType.DMA,
            pltpu.SemaphoreType.DMA,
        ],
    )(_item_body)
    itemw = item_run(item.astype(jnp.int32), iembT, w_flat)

    user_run = functools.partial(
        pl.kernel, mesh=mesh, compiler_params=cp,
        out_type=jax.ShapeDtypeStruct((B * L,), jnp.float32),
        scratch_types=[
            pltpu.VMEM((B,), jnp.int32),
            pltpu.VMEM((LCAP,), jnp.int32),
            pltpu.VMEM((LCAP,), jnp.int32),
            pltpu.VMEM((LCAP,), jnp.int32),
            pltpu.VMEM((D, WC * 128), jnp.float32),
            pltpu.VMEM((D, WC * 128), jnp.float32),
            pltpu.VMEM((PREF * D,), jnp.float32),
            pltpu.VMEM((PREF * L,), jnp.float32),
            pltpu.VMEM((L,), jnp.float32),
            pltpu.VMEM((L,), jnp.int32),
            pltpu.SemaphoreType.DMA,
            pltpu.SemaphoreType.DMA,
            pltpu.SemaphoreType.DMA,
            pltpu.SemaphoreType.DMA,
        ],
    )(_user_body)
    outw = user_run(user.astype(jnp.int32), uembT, itemw, bias16)

    return outw.reshape(B, L)[:, 0:1]


# speedup vs baseline: 1.1086x; 1.0214x over previous
import functools

import jax
import jax.numpy as jnp
from jax import lax
from jax.experimental import pallas as pl
from jax.experimental.pallas import tpu as pltpu
from jax.experimental.pallas import tpu_sc as plsc

B = 16384
D = 64
L = 16
NC = 2
NS = 16
NW = NC * NS

NB_USERS = 1000000
NB_ITEMS = 100000
UQ = (NB_USERS + 127) // 128
IQ = (NB_ITEMS + 127) // 128
UPT = (UQ + NW - 1) // NW
IPT = (IQ + NW - 1) // NW
WC = 3
UWIN = (UPT + WC - 1) // WC
IWIN = (IPT + WC - 1) // WC
NSCAN = B // L
NSUB = 8
WPS = 11
SUBW = WPS * WC
LCAP = B + L
PREF = 128

_DNUMS = lax.GatherDimensionNumbers(
    offset_dims=(), collapsed_slice_dims=(0,), start_index_map=(0,))


def _shuffle(p, perm):
    return lax.gather(p, perm, _DNUMS, (1,),
                      mode=lax.GatherScatterMode.PROMISE_IN_BOUNDS)


def _lane():
    return lax.broadcasted_iota(jnp.int32, (L,), 0)


def _lanesum(p):
    lane = _lane()
    for s in range(4):
        p = p + _shuffle(p, (lane ^ (8 >> s))[:, None])
    return p


def _splat_at(ref, e):
    return plsc.load_gather(ref, [jnp.zeros((L,), jnp.int32) + e])


def _scan_matches(idx_v, vallist, blist, lo, hi):
    lane = _lane()

    def step(i, cnt):
        v = idx_v[pl.ds(i * L, L)]
        q = v >> 7
        m = (q >= lo) & (q < hi)
        plsc.store_compressed(vallist.at[pl.ds(cnt, L)], v, mask=m)
        plsc.store_compressed(blist.at[pl.ds(cnt, L)], (i * L) + lane, mask=m)
        return cnt + jnp.max(plsc.all_reduce_population_count(m))

    return lax.fori_loop(0, NSCAN, step, jnp.int32(0))


def _window_scan(vallist, blist, winlist, cnt, wlo):
    lane = _lane()
    nv = (cnt + L - 1) // L

    def step(g, wcnt):
        v = vallist[pl.ds(g * L, L)]
        b = blist[pl.ds(g * L, L)]
        q = v >> 7
        m = (q >= wlo) & (q < wlo + WC) & ((g * L + lane) < cnt)
        packed = ((q - wlo) << 21) | ((v & 127) << 14) | b
        plsc.store_compressed(winlist.at[pl.ds(wcnt, L)], packed, mask=m)
        return wcnt + jnp.max(plsc.all_reduce_population_count(m))

    return lax.fori_loop(0, nv, step, jnp.int32(0))


def _minor_of(ps):
    return ((ps >> 21) << 7) | ((ps >> 14) & 127)


_CVEC = None


def _item_body(item_hbm, iembT_hbm, w_hbm, itemw_hbm,
               iidx_v, slist, blist, winlist, winbuf0, winbuf1, stag, w_v,
               sw0, sw1, sem_s):
    wid = lax.axis_index("s") * NC + lax.axis_index("c")
    lo = wid * IPT
    hi = jnp.minimum(lo + IPT, IQ)

    pltpu.sync_copy(item_hbm, iidx_v)
    pltpu.sync_copy(w_hbm, w_v)
    wv = tuple(w_v[pl.ds(j * L, L)] for j in range(4))
    lane = _lane()
    cvec = tuple(lane + j * L for j in range(4))

    cnt = _scan_matches(iidx_v, slist, blist, lo, hi)

    def fire_win(w, buf, sem):
        for ci in range(WC):
            q = jnp.minimum(lo + w * WC + ci, IQ - 1)
            pltpu.async_copy(
                iembT_hbm.at[:, pl.ds(pl.multiple_of(q * 128, 128), 128)],
                buf.at[:, pl.ds(ci * 128, 128)], sem)

    def process(w, buf, sem):
        wlo = lo + w * WC
        wcnt = _window_scan(slist, blist, winlist, cnt, wlo)
        npf = jnp.minimum(wcnt, PREF)
        pltpu.make_async_copy(iembT_hbm.at[:, pl.ds(0, WC * 128)], buf,
                              sem).wait()

        def fast(e, _):
            ps = _splat_at(winlist, e)
            minor = _minor_of(ps)
            for j in range(4):
                g = plsc.load_gather(buf, [cvec[j], minor]) * wv[j]
                stag[pl.ds(e * D + j * L, L)] = g
            b = jnp.max(ps) & 16383
            pltpu.async_copy(stag.at[pl.ds(e * D, D)],
                             itemw_hbm.at[pl.ds(b * D, D)], sem_s)
            return _

        lax.fori_loop(0, npf, fast, 0)

        def slow(e, _):
            ps = _splat_at(winlist, e)
            minor = _minor_of(ps)
            for j in range(4):
                g = plsc.load_gather(buf, [cvec[j], minor]) * wv[j]
                stag[pl.ds(j * L, L)] = g
            b = jnp.max(ps) & 16383
            pltpu.async_copy(stag.at[pl.ds(0, D)],
                             itemw_hbm.at[pl.ds(b * D, D)], sem_s).wait()
            return _

        lax.fori_loop(npf, wcnt, slow, 0)

        def drain(e, _):
            pltpu.make_async_copy(itemw_hbm.at[pl.ds(0, D)],
                                  stag.at[pl.ds(0, D)], sem_s).wait()
            return _

        lax.fori_loop(0, npf, drain, 0)

    fire_win(0, winbuf0, sw0)
    fire_win(1, winbuf1, sw1)

    def pair(i, _):
        process(2 * i, winbuf0, sw0)

        @pl.when(2 * i + 2 < IWIN)
        def _f0():
            fire_win(2 * i + 2, winbuf0, sw0)

        process(2 * i + 1, winbuf1, sw1)

        @pl.when(2 * i + 3 < IWIN)
        def _f1():
            fire_win(2 * i + 3, winbuf1, sw1)

        return _

    lax.fori_loop(0, IWIN // 2, pair, 0)
    if IWIN % 2:
        process(IWIN - 1, winbuf0, sw0)


def _user_body(user_hbm, uembT_hbm, itemw_hbm, bias_hbm, outw_hbm,
               uidx_v, plist, plist2, winlist, winbuf0, winbuf1, irows, vals,
               bias_v, meta_v, sw0, sw1, sem_r, sem_o):
    wid = lax.axis_index("s") * NC + lax.axis_index("c")
    lo = wid * UPT
    hi = jnp.minimum(lo + UPT, UQ)

    pltpu.sync_copy(user_hbm, uidx_v)
    pltpu.sync_copy(bias_hbm, bias_v)
    bias = bias_v[...]
    lane = _lane()
    cvec = tuple(lane + j * L for j in range(4))

    def scan_step(i, cnt):
        v = uidx_v[pl.ds(i * L, L)]
        q = v >> 7
        m = (q >= lo) & (q < hi)
        packed = ((q - lo) << 21) | ((v & 127) << 14) | ((i * L) + lane)
        plsc.store_compressed(plist.at[pl.ds(cnt, L)], packed, mask=m)
        return cnt + jnp.max(plsc.all_reduce_population_count(m))

    cnt = lax.fori_loop(0, NSCAN, scan_step, jnp.int32(0))
    nv = (cnt + L - 1) // L

    def count_step(g, counts):
        p = plist[pl.ds(g * L, L)]
        sub = (p >> 21) // SUBW
        ok = (g * L + lane) < cnt
        return tuple(
            counts[s]
            + jnp.max(plsc.all_reduce_population_count((sub == s) & ok))
            for s in range(NSUB))

    counts = lax.fori_loop(0, nv, count_step,
                           tuple(jnp.int32(0) for _ in range(NSUB)))
    offs = []
    acc = jnp.int32(0)
    for s in range(NSUB):
        offs.append(acc)
        acc = acc + counts[s]
    meta = jnp.zeros((L,), jnp.int32)
    for s in range(NSUB):
        meta = jnp.where(lane == s, offs[s], meta)
        meta = jnp.where(lane == (s + NSUB), counts[s], meta)
    meta_v[...] = meta

    def place_step(g, curs):
        p = plist[pl.ds(g * L, L)]
        sub = (p >> 21) // SUBW
        ok = (g * L + lane) < cnt
        out = []
        for s in range(NSUB):
            m = (sub == s) & ok
            plsc.store_compressed(plist2.at[pl.ds(curs[s], L)], p, mask=m)
            out.append(curs[s]
                       + jnp.max(plsc.all_reduce_population_count(m)))
        return tuple(out)

    lax.fori_loop(0, nv, place_step, tuple(offs))

    def fire_win(w, buf, sem):
        for ci in range(WC):
            q = jnp.minimum(lo + w * WC + ci, UQ - 1)
            pltpu.async_copy(
                uembT_hbm.at[:, pl.ds(pl.multiple_of(q * 128, 128), 128)],
                buf.at[:, pl.ds(ci * 128, 128)], sem)

    def window_scan_sub(w):
        s = w // WPS
        off_s = jnp.max(_splat_at(meta_v, s))
        cnt_s = jnp.max(_splat_at(meta_v, s + NSUB))
        wrel = w * WC
        nvs = (cnt_s + L - 1) // L

        def step(g, wcnt):
            p = plist2[pl.ds(off_s + g * L, L)]
            qr = p >> 21
            m = ((qr >= wrel) & (qr < wrel + WC)
                 & ((g * L + lane) < cnt_s))
            packed = ((qr - wrel) << 21) | (p & 2097151)
            plsc.store_compressed(winlist.at[pl.ds(wcnt, L)], packed, mask=m)
            return wcnt + jnp.max(plsc.all_reduce_population_count(m))

        return lax.fori_loop(0, nvs, step, jnp.int32(0))

    def pref_one(e):
        ps = _splat_at(winlist, e)
        b = jnp.max(ps) & 16383
        pltpu.async_copy(itemw_hbm.at[pl.ds(b * D, D)],
                         irows.at[pl.ds(e * D, D)], sem_r)

    def one(e, buf):
        ps = _splat_at(winlist, e)
        minor = _minor_of(ps)
        acc = None
        for j in range(4):
            t = (plsc.load_gather(buf, [cvec[j], minor])
                 * irows[pl.ds(e * D + j * L, L)])
            acc = t if acc is None else acc + t
        val = _lanesum(acc) + bias
        vals[pl.ds(e * L, L)] = val
        b = jnp.max(ps) & 16383
        pltpu.async_copy(vals.at[pl.ds(e * L, L)],
                         outw_hbm.at[pl.ds(b * L, L)], sem_o)

    def process(w, buf, sem):
        wcnt = window_scan_sub(w)
        npf = jnp.minimum(wcnt, PREF)

        def pref(e, _):
            pref_one(e)
            return _

        lax.fori_loop(0, npf, pref, 0)
        pltpu.make_async_copy(uembT_hbm.at[:, pl.ds(0, WC * 128)], buf,
                              sem).wait()

        def drain_r(e, _):
            pltpu.make_async_copy(itemw_hbm.at[pl.ds(0, D)],
                                  irows.at[pl.ds(0, D)], sem_r).wait()
            return _

        lax.fori_loop(0, npf, drain_r, 0)

        def fast(e, _):
            one(e, buf)
            return _

        lax.fori_loop(0, npf, fast, 0)

        def slow(e, _):
            ps = _splat_at(winlist, e)
            minor = _minor_of(ps)
            b = jnp.max(ps) & 16383
            pltpu.async_copy(itemw_hbm.at[pl.ds(b * D, D)],
                             irows.at[pl.ds(0, D)], sem_r).wait()
            acc = None
            for j in range(4):
                t = (plsc.load_gather(buf, [cvec[j], minor])
                     * irows[pl.ds(j * L, L)])
                acc = t if acc is None else acc + t
            val = _lanesum(acc) + bias
            vals[pl.ds(0, L)] = val
            pltpu.async_copy(vals.at[pl.ds(0, L)],
                             outw_hbm.at[pl.ds(b * L, L)], sem_o).wait()
            return _

        lax.fori_loop(npf, wcnt, slow, 0)

        def drain_o(e, _):
            pltpu.make_async_copy(outw_hbm.at[pl.ds(0, L)],
                                  vals.at[pl.ds(0, L)], sem_o).wait()
            return _

        lax.fori_loop(0, npf, drain_o, 0)

    fire_win(0, winbuf0, sw0)
    fire_win(1, winbuf1, sw1)

    def pair(i, _):
        process(2 * i, winbuf0, sw0)

        @pl.when(2 * i + 2 < UWIN)
        def _f0():
            fire_win(2 * i + 2, winbuf0, sw0)

        process(2 * i + 1, winbuf1, sw1)

        @pl.when(2 * i + 3 < UWIN)
        def _f1():
            fire_win(2 * i + 3, winbuf1, sw1)

        return _

    lax.fori_loop(0, (UWIN + 1) // 2, pair, 0)


@jax.jit
def kernel(user, item, mf_user_embed, mf_item_embed, final_w, final_b):
    uembT = mf_user_embed.T
    iembT = mf_item_embed.T
    w_flat = final_w.reshape(D)
    bias16 = jnp.tile(final_b.reshape(1), L)
    mesh = plsc.VectorSubcoreMesh(core_axis_name="c", subcore_axis_name="s")
    cp = pltpu.CompilerParams(use_tc_tiling_on_sc=True,
                              needs_layout_passes=False)

    item_run = functools.partial(
        pl.kernel, mesh=mesh, compiler_params=cp,
        out_type=jax.ShapeDtypeStruct((B * D,), jnp.float32),
        scratch_types=[
            pltpu.VMEM((B,), jnp.int32),
            pltpu.VMEM((LCAP,), jnp.int32),
            pltpu.VMEM((LCAP,), jnp.int32),
            pltpu.VMEM((LCAP,), jnp.int32),
            pltpu.VMEM((D, WC * 128), jnp.float32),
            pltpu.VMEM((D, WC * 128), jnp.float32),
            pltpu.VMEM((PREF * D,), jnp.float32),
            pltpu.VMEM((D,), jnp.float32),
            pltpu.SemaphoreType.DMA,
            pltpu.SemaphoreType.DMA,
            pltpu.SemaphoreType.DMA,
        ],
    )(_item_body)
    itemw = item_run(item.astype(jnp.int32), iembT, w_flat)

    user_run = functools.partial(
        pl.kernel, mesh=mesh, compiler_params=cp,
        out_type=jax.ShapeDtypeStruct((B * L,), jnp.float32),
        scratch_types=[
            pltpu.VMEM((B,), jnp.int32),
            pltpu.VMEM((LCAP,), jnp.int32),
            pltpu.VMEM((LCAP,), jnp.int32),
            pltpu.VMEM((LCAP,), jnp.int32),
            pltpu.VMEM((D, WC * 128), jnp.float32),
            pltpu.VMEM((D, WC * 128), jnp.float32),
            pltpu.VMEM((PREF * D,), jnp.float32),
            pltpu.VMEM((PREF * L,), jnp.float32),
            pltpu.VMEM((L,), jnp.float32),
            pltpu.VMEM((L,), jnp.int32),
            pltpu.SemaphoreType.DMA,
            pltpu.SemaphoreType.DMA,
            pltpu.SemaphoreType.DMA,
            pltpu.SemaphoreType.DMA,
        ],
    )(_user_body)
    outw = user_run(user.astype(jnp.int32), uembT, itemw, bias16)

    return outw.reshape(B, L)[:, 0:1]


# speedup vs baseline: 1.1689x; 1.0543x over previous
import functools

import jax
import jax.numpy as jnp
from jax import lax
from jax.experimental import pallas as pl
from jax.experimental.pallas import tpu as pltpu
from jax.experimental.pallas import tpu_sc as plsc

B = 16384
D = 64
L = 16
NC = 2
NS = 16
NW = NC * NS

NB_USERS = 1000000
NB_ITEMS = 100000
UQ = (NB_USERS + 127) // 128
IQ = (NB_ITEMS + 127) // 128
UPT = (UQ + NW - 1) // NW
IPT = (IQ + NW - 1) // NW
WC = 3
UWC = 4
UWIN = (UPT + UWC - 1) // UWC
IWIN = (IPT + WC - 1) // WC
NSCAN = B // L
NSUB = 8
WPS = 8
SUBW = WPS * UWC
LCAP = B + L
PREF = 128

_DNUMS = lax.GatherDimensionNumbers(
    offset_dims=(), collapsed_slice_dims=(0,), start_index_map=(0,))


def _shuffle(p, perm):
    return lax.gather(p, perm, _DNUMS, (1,),
                      mode=lax.GatherScatterMode.PROMISE_IN_BOUNDS)


def _lane():
    return lax.broadcasted_iota(jnp.int32, (L,), 0)


def _lanesum(p):
    lane = _lane()
    for s in range(4):
        p = p + _shuffle(p, (lane ^ (8 >> s))[:, None])
    return p


def _splat_at(ref, e):
    return plsc.load_gather(ref, [jnp.zeros((L,), jnp.int32) + e])


def _scan_matches(idx_v, vallist, blist, lo, hi):
    lane = _lane()

    def step(i, cnt):
        v = idx_v[pl.ds(i * L, L)]
        q = v >> 7
        m = (q >= lo) & (q < hi)
        plsc.store_compressed(vallist.at[pl.ds(cnt, L)], v, mask=m)
        plsc.store_compressed(blist.at[pl.ds(cnt, L)], (i * L) + lane, mask=m)
        return cnt + jnp.max(plsc.all_reduce_population_count(m))

    return lax.fori_loop(0, NSCAN, step, jnp.int32(0))


def _window_scan(vallist, blist, winlist, cnt, wlo):
    lane = _lane()
    nv = (cnt + L - 1) // L

    def step(g, wcnt):
        v = vallist[pl.ds(g * L, L)]
        b = blist[pl.ds(g * L, L)]
        q = v >> 7
        m = (q >= wlo) & (q < wlo + WC) & ((g * L + lane) < cnt)
        packed = ((q - wlo) << 21) | ((v & 127) << 14) | b
        plsc.store_compressed(winlist.at[pl.ds(wcnt, L)], packed, mask=m)
        return wcnt + jnp.max(plsc.all_reduce_population_count(m))

    return lax.fori_loop(0, nv, step, jnp.int32(0))


def _minor_of(ps):
    return ((ps >> 21) << 7) | ((ps >> 14) & 127)


_CVEC = None


def _item_body(item_hbm, iembT_hbm, w_hbm, itemw_hbm,
               iidx_v, slist, blist, winlist, winbuf0, winbuf1, stag, w_v,
               sw0, sw1, sem_s):
    wid = lax.axis_index("s") * NC + lax.axis_index("c")
    lo = wid * IPT
    hi = jnp.minimum(lo + IPT, IQ)

    pltpu.sync_copy(item_hbm, iidx_v)
    pltpu.sync_copy(w_hbm, w_v)
    wv = tuple(w_v[pl.ds(j * L, L)] for j in range(4))
    lane = _lane()
    cvec = tuple(lane + j * L for j in range(4))

    cnt = _scan_matches(iidx_v, slist, blist, lo, hi)

    def fire_win(w, buf, sem):
        for ci in range(WC):
            q = jnp.minimum(lo + w * WC + ci, IQ - 1)
            pltpu.async_copy(
                iembT_hbm.at[:, pl.ds(pl.multiple_of(q * 128, 128), 128)],
                buf.at[:, pl.ds(ci * 128, 128)], sem)

    def process(w, buf, sem):
        wlo = lo + w * WC
        wcnt = _window_scan(slist, blist, winlist, cnt, wlo)
        npf = jnp.minimum(wcnt, PREF)
        pltpu.make_async_copy(iembT_hbm.at[:, pl.ds(0, WC * 128)], buf,
                              sem).wait()

        def fast(e, _):
            ps = _splat_at(winlist, e)
            minor = _minor_of(ps)
            for j in range(4):
                g = plsc.load_gather(buf, [cvec[j], minor]) * wv[j]
                stag[pl.ds(e * D + j * L, L)] = g
            b = jnp.max(ps) & 16383
            pltpu.async_copy(stag.at[pl.ds(e * D, D)],
                             itemw_hbm.at[pl.ds(b * D, D)], sem_s)
            return _

        lax.fori_loop(0, npf, fast, 0)

        def slow(e, _):
            ps = _splat_at(winlist, e)
            minor = _minor_of(ps)
            for j in range(4):
                g = plsc.load_gather(buf, [cvec[j], minor]) * wv[j]
                stag[pl.ds(j * L, L)] = g
            b = jnp.max(ps) & 16383
            pltpu.async_copy(stag.at[pl.ds(0, D)],
                             itemw_hbm.at[pl.ds(b * D, D)], sem_s).wait()
            return _

        lax.fori_loop(npf, wcnt, slow, 0)

        def drain(e, _):
            pltpu.make_async_copy(itemw_hbm.at[pl.ds(0, D)],
                                  stag.at[pl.ds(0, D)], sem_s).wait()
            return _

        lax.fori_loop(0, npf, drain, 0)

    fire_win(0, winbuf0, sw0)
    fire_win(1, winbuf1, sw1)

    def pair(i, _):
        process(2 * i, winbuf0, sw0)

        @pl.when(2 * i + 2 < IWIN)
        def _f0():
            fire_win(2 * i + 2, winbuf0, sw0)

        process(2 * i + 1, winbuf1, sw1)

        @pl.when(2 * i + 3 < IWIN)
        def _f1():
            fire_win(2 * i + 3, winbuf1, sw1)

        return _

    lax.fori_loop(0, IWIN // 2, pair, 0)
    if IWIN % 2:
        process(IWIN - 1, winbuf0, sw0)


def _user_body(user_hbm, uembT_hbm, itemw_hbm, bias_hbm, outw_hbm,
               uidx_v, plist, plist2, winbuf0, winbuf1, irows, vals,
               bias_v, meta_v, sw0, sw1, sem_r, sem_o):
    wid = lax.axis_index("s") * NC + lax.axis_index("c")
    lo = wid * UPT
    hi = jnp.minimum(lo + UPT, UQ)

    pltpu.sync_copy(user_hbm, uidx_v)
    pltpu.sync_copy(bias_hbm, bias_v)
    bias = bias_v[...]
    lane = _lane()
    cvec = tuple(lane + j * L for j in range(4))

    def scan_step(i, cnt):
        v = uidx_v[pl.ds(i * L, L)]
        q = v >> 7
        m = (q >= lo) & (q < hi)
        packed = ((q - lo) << 21) | ((v & 127) << 14) | ((i * L) + lane)
        plsc.store_compressed(plist.at[pl.ds(cnt, L)], packed, mask=m)
        return cnt + jnp.max(plsc.all_reduce_population_count(m))

    cnt = lax.fori_loop(0, NSCAN, scan_step, jnp.int32(0))
    nv = (cnt + L - 1) // L

    def count_step(g, counts):
        p = plist[pl.ds(g * L, L)]
        sub = (p >> 21) // SUBW
        ok = (g * L + lane) < cnt
        return tuple(
            counts[s]
            + jnp.max(plsc.all_reduce_population_count((sub == s) & ok))
            for s in range(NSUB))

    counts = lax.fori_loop(0, nv, count_step,
                           tuple(jnp.int32(0) for _ in range(NSUB)))
    offs = []
    acc = jnp.int32(0)
    for s in range(NSUB):
        offs.append(acc)
        acc = acc + counts[s]
    meta = jnp.zeros((L,), jnp.int32)
    for s in range(NSUB):
        meta = jnp.where(lane == s, offs[s], meta)
        meta = jnp.where(lane == (s + NSUB), counts[s], meta)
    meta_v[...] = meta

    def place_step(g, curs):
        p = plist[pl.ds(g * L, L)]
        sub = (p >> 21) // SUBW
        ok = (g * L + lane) < cnt
        out = []
        for s in range(NSUB):
            m = (sub == s) & ok
            plsc.store_compressed(plist2.at[pl.ds(curs[s], L)], p, mask=m)
            out.append(curs[s]
                       + jnp.max(plsc.all_reduce_population_count(m)))
        return tuple(out)

    lax.fori_loop(0, nv, place_step, tuple(offs))

    def fire_win(w, buf, sem):
        for ci in range(UWC):
            q = jnp.minimum(lo + w * UWC + ci, UQ - 1)
            pltpu.async_copy(
                uembT_hbm.at[:, pl.ds(pl.multiple_of(q * 128, 128), 128)],
                buf.at[:, pl.ds(ci * 128, 128)], sem)

    def window_scan_sub(w):
        s = w // WPS
        off_s = jnp.max(_splat_at(meta_v, s))
        cnt_s = jnp.max(_splat_at(meta_v, s + NSUB))
        wrel = w * UWC
        nvs = (cnt_s + L - 1) // L

        def step(g, wcnt):
            p = plist2[pl.ds(off_s + g * L, L)]
            qr = p >> 21
            m = ((qr >= wrel) & (qr < wrel + UWC)
                 & ((g * L + lane) < cnt_s))
            packed = ((qr - wrel) << 21) | (p & 2097151)
            plsc.store_compressed(plist.at[pl.ds(wcnt, L)], packed, mask=m)
            return wcnt + jnp.max(plsc.all_reduce_population_count(m))

        return lax.fori_loop(0, nvs, step, jnp.int32(0))

    def pref_one(e):
        ps = _splat_at(plist, e)
        b = jnp.max(ps) & 16383
        pltpu.async_copy(itemw_hbm.at[pl.ds(b * D, D)],
                         irows.at[pl.ds(e * D, D)], sem_r)

    def one(e, buf):
        ps = _splat_at(plist, e)
        minor = _minor_of(ps)
        acc = None
        for j in range(4):
            t = (plsc.load_gather(buf, [cvec[j], minor])
                 * irows[pl.ds(e * D + j * L, L)])
            acc = t if acc is None else acc + t
        val = _lanesum(acc) + bias
        vals[pl.ds(e * L, L)] = val
        b = jnp.max(ps) & 16383
        pltpu.async_copy(vals.at[pl.ds(e * L, L)],
                         outw_hbm.at[pl.ds(b * L, L)], sem_o)

    def process(w, buf, sem):
        wcnt = window_scan_sub(w)
        npf = jnp.minimum(wcnt, PREF)

        def pref(e, _):
            pref_one(e)
            return _

        lax.fori_loop(0, npf, pref, 0)
        pltpu.make_async_copy(uembT_hbm.at[:, pl.ds(0, UWC * 128)], buf,
                              sem).wait()

        def drain_r(e, _):
            pltpu.make_async_copy(itemw_hbm.at[pl.ds(0, D)],
                                  irows.at[pl.ds(0, D)], sem_r).wait()
            return _

        lax.fori_loop(0, npf, drain_r, 0)

        def fast(e, _):
            one(e, buf)
            return _

        lax.fori_loop(0, npf, fast, 0)

        def slow(e, _):
            ps = _splat_at(plist, e)
            minor = _minor_of(ps)
            b = jnp.max(ps) & 16383
            pltpu.async_copy(itemw_hbm.at[pl.ds(b * D, D)],
                             irows.at[pl.ds(0, D)], sem_r).wait()
            acc = None
            for j in range(4):
                t = (plsc.load_gather(buf, [cvec[j], minor])
                     * irows[pl.ds(j * L, L)])
                acc = t if acc is None else acc + t
            val = _lanesum(acc) + bias
            vals[pl.ds(0, L)] = val
            pltpu.async_copy(vals.at[pl.ds(0, L)],
                             outw_hbm.at[pl.ds(b * L, L)], sem_o).wait()
            return _

        lax.fori_loop(npf, wcnt, slow, 0)

        def drain_o(e, _):
            pltpu.make_async_copy(outw_hbm.at[pl.ds(0, L)],
                                  vals.at[pl.ds(0, L)], sem_o).wait()
            return _

        lax.fori_loop(0, npf, drain_o, 0)

    fire_win(0, winbuf0, sw0)
    fire_win(1, winbuf1, sw1)

    def pair(i, _):
        process(2 * i, winbuf0, sw0)

        @pl.when(2 * i + 2 < UWIN)
        def _f0():
            fire_win(2 * i + 2, winbuf0, sw0)

        process(2 * i + 1, winbuf1, sw1)

        @pl.when(2 * i + 3 < UWIN)
        def _f1():
            fire_win(2 * i + 3, winbuf1, sw1)

        return _

    lax.fori_loop(0, (UWIN + 1) // 2, pair, 0)


@jax.jit
def kernel(user, item, mf_user_embed, mf_item_embed, final_w, final_b):
    uembT = mf_user_embed.T
    iembT = mf_item_embed.T
    w_flat = final_w.reshape(D)
    bias16 = jnp.tile(final_b.reshape(1), L)
    mesh = plsc.VectorSubcoreMesh(core_axis_name="c", subcore_axis_name="s")
    cp = pltpu.CompilerParams(use_tc_tiling_on_sc=True,
                              needs_layout_passes=False)

    item_run = functools.partial(
        pl.kernel, mesh=mesh, compiler_params=cp,
        out_type=jax.ShapeDtypeStruct((B * D,), jnp.float32),
        scratch_types=[
            pltpu.VMEM((B,), jnp.int32),
            pltpu.VMEM((LCAP,), jnp.int32),
            pltpu.VMEM((LCAP,), jnp.int32),
            pltpu.VMEM((LCAP,), jnp.int32),
            pltpu.VMEM((D, WC * 128), jnp.float32),
            pltpu.VMEM((D, WC * 128), jnp.float32),
            pltpu.VMEM((PREF * D,), jnp.float32),
            pltpu.VMEM((D,), jnp.float32),
            pltpu.SemaphoreType.DMA,
            pltpu.SemaphoreType.DMA,
            pltpu.SemaphoreType.DMA,
        ],
    )(_item_body)
    itemw = item_run(item.astype(jnp.int32), iembT, w_flat)

    user_run = functools.partial(
        pl.kernel, mesh=mesh, compiler_params=cp,
        out_type=jax.ShapeDtypeStruct((B * L,), jnp.float32),
        scratch_types=[
            pltpu.VMEM((B,), jnp.int32),
            pltpu.VMEM((LCAP,), jnp.int32),
            pltpu.VMEM((LCAP,), jnp.int32),
            pltpu.VMEM((D, UWC * 128), jnp.float32),
            pltpu.VMEM((D, UWC * 128), jnp.float32),
            pltpu.VMEM((PREF * D,), jnp.float32),
            pltpu.VMEM((PREF * L,), jnp.float32),
            pltpu.VMEM((L,), jnp.float32),
            pltpu.VMEM((L,), jnp.int32),
            pltpu.SemaphoreType.DMA,
            pltpu.SemaphoreType.DMA,
            pltpu.SemaphoreType.DMA,
            pltpu.SemaphoreType.DMA,
        ],
    )(_user_body)
    outw = user_run(user.astype(jnp.int32), uembT, itemw, bias16)

    return outw.reshape(B, L)[:, 0:1]
